# baseline (speedup 1.0000x reference)
import functools

import jax
import jax.numpy as jnp
from jax import lax
from jax.experimental import pallas as pl
from jax.experimental.pallas import tpu as pltpu
from jax.experimental.pallas import tpu_sc as plsc

B = 16384
DG = 16
DM = 64

_info = plsc.get_sparse_core_info()
NC = _info.num_cores
NS = _info.num_subcores
NW = NC * NS
RPW = B // NW
CH = 128
NCH = RPW // CH


@functools.partial(
    pl.kernel,
    out_type=(
        jax.ShapeDtypeStruct((B, DM), jnp.float32),
        jax.ShapeDtypeStruct((B, DM), jnp.float32),
        jax.ShapeDtypeStruct((B, DG), jnp.float32),
        jax.ShapeDtypeStruct((B, DG), jnp.float32),
    ),
    mesh=plsc.VectorSubcoreMesh(core_axis_name="c", subcore_axis_name="s"),
    compiler_params=pltpu.CompilerParams(use_tc_tiling_on_sc=False),
    scratch_types=[
        pltpu.VMEM((NCH, CH), jnp.int32),
        pltpu.VMEM((NCH, CH), jnp.int32),
        pltpu.VMEM((RPW, DM), jnp.float32),
        pltpu.VMEM((RPW, DM), jnp.float32),
        pltpu.VMEM((RPW, DG), jnp.float32),
        pltpu.VMEM((RPW, DG), jnp.float32),
        pltpu.SemaphoreType.DMA,
        pltpu.SemaphoreType.DMA,
        pltpu.SemaphoreType.DMA,
        pltpu.SemaphoreType.DMA,
    ],
)
def _sc_gather(user_hbm, item_hbm, eum_hbm, eim_hbm, eug_hbm, eig_hbm,
               um_out, im_out, ug_out, ig_out,
               uidx, iidx, um_v, im_v, ug_v, ig_v, s0, s1, s2, s3):
    wid = lax.axis_index("s") * NC + lax.axis_index("c")
    base = wid * RPW
    pltpu.sync_copy(user_hbm.at[wid], uidx)
    pltpu.sync_copy(item_hbm.at[wid], iidx)
    cps = []
    for j in range(NCH):
        sl = pl.ds(j * CH, CH)
        cps.append(pltpu.async_copy(eum_hbm.at[uidx.at[j]], um_v.at[sl], s0))
        cps.append(pltpu.async_copy(eim_hbm.at[iidx.at[j]], im_v.at[sl], s1))
        cps.append(pltpu.async_copy(eug_hbm.at[uidx.at[j]], ug_v.at[sl], s2))
        cps.append(pltpu.async_copy(eig_hbm.at[iidx.at[j]], ig_v.at[sl], s3))
    for cp in cps:
        cp.wait()
    pltpu.sync_copy(um_v, um_out.at[pl.ds(base, RPW)])
    pltpu.sync_copy(im_v, im_out.at[pl.ds(base, RPW)])
    pltpu.sync_copy(ug_v, ug_out.at[pl.ds(base, RPW)])
    pltpu.sync_copy(ig_v, ig_out.at[pl.ds(base, RPW)])


def _dense_body(um_ref, im_ref, ug_ref, ig_ref, w1u_ref, w1i_ref, b1_ref,
                w2_ref, b2_ref, w3_ref, b3_ref, wpg_ref, wph_ref, bp_ref,
                o_ref):
    h = um_ref[...] @ w1u_ref[...] + im_ref[...] @ w1i_ref[...] + b1_ref[...]
    h = jnp.maximum(h, 0.0)
    h = jnp.maximum(h @ w2_ref[...] + b2_ref[...], 0.0)
    h = jnp.maximum(h @ w3_ref[...] + b3_ref[...], 0.0)
    gmf = ug_ref[...] * ig_ref[...]
    z = gmf @ wpg_ref[...] + h @ wph_ref[...] + bp_ref[...]
    o_ref[...] = 1.0 / (1.0 + jnp.exp(-z))


def _tc_dense(um, im, ug, ig, w1u, w1i, b1, w2, b2, w3, b3, wpg, wph, bp):
    BLK = 2048
    row = lambda i: (i, 0)
    rep = lambda i: (0, 0)
    return pl.pallas_call(
        _dense_body,
        grid=(B // BLK,),
        in_specs=[
            pl.BlockSpec((BLK, DM), row),
            pl.BlockSpec((BLK, DM), row),
            pl.BlockSpec((BLK, DG), row),
            pl.BlockSpec((BLK, DG), row),
            pl.BlockSpec((DM, DM), rep),
            pl.BlockSpec((DM, DM), rep),
            pl.BlockSpec((1, DM), rep),
            pl.BlockSpec((DM, DM // 2), rep),
            pl.BlockSpec((1, DM // 2), rep),
            pl.BlockSpec((DM // 2, DG), rep),
            pl.BlockSpec((1, DG), rep),
            pl.BlockSpec((DG, 1), rep),
            pl.BlockSpec((DG, 1), rep),
            pl.BlockSpec((1, 1), rep),
        ],
        out_specs=pl.BlockSpec((BLK, 1), row),
        out_shape=jax.ShapeDtypeStruct((B, 1), jnp.float32),
    )(um, im, ug, ig, w1u, w1i, b1, w2, b2, w3, b3, wpg, wph, bp)


def kernel(user, item, embed_user_GMF, embed_item_GMF, embed_user_MLP,
           embed_item_MLP, W1, b1, W2, b2, W3, b3, Wp, bp):
    user3 = user.astype(jnp.int32).reshape(NW, NCH, CH)
    item3 = item.astype(jnp.int32).reshape(NW, NCH, CH)
    um, im, ug, ig = _sc_gather(user3, item3, embed_user_MLP, embed_item_MLP,
                                embed_user_GMF, embed_item_GMF)
    out = _tc_dense(
        um, im, ug, ig,
        W1[:DM], W1[DM:], b1.reshape(1, DM),
        W2, b2.reshape(1, DM // 2),
        W3, b3.reshape(1, DG),
        Wp[:DG], Wp[DG:], bp.reshape(1, 1),
    )
    return out.reshape(-1)
